# Initial kernel scaffold; baseline (speedup 1.0000x reference)
import jax
import jax.numpy as jnp
from jax.experimental import pallas as pl


def kernel(mllm_mask, byt5_mask):
    raise NotImplementedError("write your pallas kernel here")


# speedup vs baseline: 1.0963x; 1.0963x over previous
import jax
import jax.numpy as jnp
from jax import lax
from jax.experimental import pallas as pl
from jax.experimental.pallas import tpu as pltpu, tpu_sc as plsc

_N = 1985
_NPAD = 2048
_L = 16
_NV = _NPAD // _L


def _sc_body(c_hbm, idx_hbm, zm_hbm, c_v, idx_v, zm_v):
    is_lead = jnp.logical_and(lax.axis_index("c") == 0, lax.axis_index("s") == 0)

    @pl.when(is_lead)
    def _():
        pltpu.sync_copy(c_hbm, c_v)

        def sum_body(k, acc):
            return acc + c_v[pl.ds(k * _L, _L)]

        acc = lax.fori_loop(0, _NV, sum_body, jnp.zeros((_L,), jnp.float32))
        n_valid = jnp.sum(acc)

        lane = lax.iota(jnp.int32, _L)

        def part_body(k, ones_before):
            c = c_v[pl.ds(k * _L, _L)]
            incl = plsc.cumsum(c)
            one_excl = ones_before + incl - c
            gi = k * _L + lane
            gi_f = gi.astype(jnp.float32)
            pos_f = jnp.where(c >= 0.5, one_excl, n_valid + gi_f - one_excl)
            plsc.store_scatter(idx_v, [pos_f.astype(jnp.int32)], gi)
            zm_v[pl.ds(k * _L, _L)] = (gi_f < n_valid).astype(jnp.float32)
            return ones_before + jnp.sum(c)

        lax.fori_loop(0, _NV, part_body, jnp.float32(0.0))

        pltpu.sync_copy(idx_v, idx_hbm)
        pltpu.sync_copy(zm_v, zm_hbm)


_sc_call = pl.kernel(
    _sc_body,
    out_type=(
        jax.ShapeDtypeStruct((_NPAD,), jnp.int32),
        jax.ShapeDtypeStruct((_NPAD,), jnp.float32),
    ),
    mesh=plsc.VectorSubcoreMesh(core_axis_name="c", subcore_axis_name="s"),
    scratch_types=(
        pltpu.VMEM((_NPAD,), jnp.float32),
        pltpu.VMEM((_NPAD,), jnp.int32),
        pltpu.VMEM((_NPAD,), jnp.float32),
    ),
    compiler_params=pltpu.CompilerParams(needs_layout_passes=False),
)


def kernel(mllm_mask, byt5_mask):
    combined = jnp.concatenate([
        mllm_mask.astype(jnp.float32),
        byt5_mask.astype(jnp.float32),
        jnp.zeros((_NPAD - 1256,), jnp.float32),
    ])
    idx_pad, zm_pad = _sc_call(combined)
    return idx_pad[:_N], zm_pad[:_N]


# speedup vs baseline: 1.1012x; 1.0044x over previous
import jax
import jax.numpy as jnp
from jax import lax
from jax.experimental import pallas as pl
from jax.experimental.pallas import tpu as pltpu, tpu_sc as plsc

_N = 1985
_NM = 1256
_L = 16
_NP = 1280
_NVP = _NP // _L
_NVF = 2048 // _L


def _sc_body(mllm_hbm, byt5_hbm, idx_hbm, zm_hbm, c_v, idx_v, zm_v):
    is_lead = jnp.logical_and(lax.axis_index("c") == 0, lax.axis_index("s") == 0)

    @pl.when(is_lead)
    def _():
        pltpu.sync_copy(mllm_hbm, c_v.at[pl.ds(0, 1000)])
        pltpu.sync_copy(byt5_hbm, c_v.at[pl.ds(1000, 256)])
        zeros = jnp.zeros((_L,), jnp.float32)
        c_v[pl.ds(_NM, _L)] = zeros
        c_v[pl.ds(_NP - _L, _L)] = zeros

        def sum_body(k, acc):
            return acc + c_v[pl.ds(k * _L, _L)]

        acc = lax.fori_loop(0, _NVP, sum_body, jnp.zeros((_L,), jnp.float32))
        n_valid = jnp.sum(acc)

        lane = lax.iota(jnp.int32, _L)
        lane_f = lane.astype(jnp.float32)

        def part_body(k, ones_before):
            c = c_v[pl.ds(k * _L, _L)]
            incl = plsc.cumsum(c)
            one_excl = ones_before + incl - c
            gi = k * _L + lane
            gi_f = gi.astype(jnp.float32)
            pos_f = jnp.where(c >= 0.5, one_excl, n_valid + gi_f - one_excl)
            plsc.store_scatter(idx_v, [pos_f.astype(jnp.int32)], gi)
            zm_v[pl.ds(k * _L, _L)] = (gi_f < n_valid).astype(jnp.float32)
            return ones_before + jnp.sum(c)

        lax.fori_loop(0, _NVP, part_body, jnp.float32(0.0))

        def fill_body(k, _):
            idx_v[pl.ds(k * _L, _L)] = k * _L + lane
            zm_v[pl.ds(k * _L, _L)] = zeros
            return 0

        lax.fori_loop(_NVP, _NVF, fill_body, 0)

        pltpu.sync_copy(idx_v.at[pl.ds(0, 1984)], idx_hbm.at[pl.ds(0, 1984)])
        pltpu.sync_copy(idx_v.at[pl.ds(1984, 1)], idx_hbm.at[pl.ds(1984, 1)])
        pltpu.sync_copy(zm_v.at[pl.ds(0, 1984)], zm_hbm.at[pl.ds(0, 1984)])
        pltpu.sync_copy(zm_v.at[pl.ds(1984, 1)], zm_hbm.at[pl.ds(1984, 1)])


_sc_call = pl.kernel(
    _sc_body,
    out_type=(
        jax.ShapeDtypeStruct((_N,), jnp.int32),
        jax.ShapeDtypeStruct((_N,), jnp.float32),
    ),
    mesh=plsc.VectorSubcoreMesh(core_axis_name="c", subcore_axis_name="s"),
    scratch_types=(
        pltpu.VMEM((_NP,), jnp.float32),
        pltpu.VMEM((2048,), jnp.int32),
        pltpu.VMEM((2048,), jnp.float32),
    ),
    compiler_params=pltpu.CompilerParams(needs_layout_passes=False),
)


def kernel(mllm_mask, byt5_mask):
    return _sc_call(mllm_mask, byt5_mask)


# speedup vs baseline: 1.1737x; 1.0659x over previous
import jax
import jax.numpy as jnp
from jax import lax
from jax.experimental import pallas as pl
from jax.experimental.pallas import tpu as pltpu, tpu_sc as plsc

_N = 1985
_NM = 1256
_L = 16
_NP = 1280
_NVP = _NP // _L
_NVF = 2048 // _L


def _sc_body(mllm_hbm, byt5_hbm, idx_hbm, zm_hbm, c_v, idx_v, zm_v):
    is_lead = jnp.logical_and(lax.axis_index("c") == 0, lax.axis_index("s") == 0)

    @pl.when(is_lead)
    def _():
        pltpu.sync_copy(mllm_hbm, c_v.at[pl.ds(0, 1000)])
        pltpu.sync_copy(byt5_hbm, c_v.at[pl.ds(1000, 256)])
        zeros = jnp.zeros((_L,), jnp.float32)
        c_v[pl.ds(_NM, _L)] = zeros
        c_v[pl.ds(_NP - _L, _L)] = zeros

        def sum_body(k, acc):
            return acc + c_v[pl.ds(k * _L, _L)]

        acc = lax.fori_loop(0, _NVP, sum_body, jnp.zeros((_L,), jnp.float32))
        n_valid = jnp.sum(acc)

        lane = lax.iota(jnp.int32, _L)
        lane_f = lane.astype(jnp.float32)

        def part_body(k, ones_before):
            c = c_v[pl.ds(k * _L, _L)]
            incl = plsc.cumsum(c)
            one_excl = ones_before + incl - c
            gi = k * _L + lane
            gi_f = gi.astype(jnp.float32)
            pos_f = jnp.where(c >= 0.5, one_excl, n_valid + gi_f - one_excl)
            plsc.store_scatter(idx_v, [pos_f.astype(jnp.int32)], gi)
            zm_v[pl.ds(k * _L, _L)] = (gi_f < n_valid).astype(jnp.float32)
            return ones_before + jnp.sum(c)

        lax.fori_loop(0, _NVP, part_body, jnp.float32(0.0))

        def fill_body(k, _):
            idx_v[pl.ds(k * _L, _L)] = k * _L + lane
            zm_v[pl.ds(k * _L, _L)] = zeros
            return 0

        lax.fori_loop(_NVP, _NVF, fill_body, 0)

        pltpu.sync_copy(idx_v.at[pl.ds(0, 1984)], idx_hbm.at[pl.ds(0, 1984)])
        pltpu.sync_copy(idx_v.at[pl.ds(1984, 1)], idx_hbm.at[pl.ds(1984, 1)])
        pltpu.sync_copy(zm_v.at[pl.ds(0, 1984)], zm_hbm.at[pl.ds(0, 1984)])
        pltpu.sync_copy(zm_v.at[pl.ds(1984, 1)], zm_hbm.at[pl.ds(1984, 1)])


_sc_call = pl.kernel(
    _sc_body,
    out_type=(
        jax.ShapeDtypeStruct((_N,), jnp.int32),
        jax.ShapeDtypeStruct((_N,), jnp.float32),
    ),
    mesh=plsc.VectorSubcoreMesh(core_axis_name="c", subcore_axis_name="s", num_cores=1),
    scratch_types=(
        pltpu.VMEM((_NP,), jnp.float32),
        pltpu.VMEM((2048,), jnp.int32),
        pltpu.VMEM((2048,), jnp.float32),
    ),
    compiler_params=pltpu.CompilerParams(needs_layout_passes=False),
)


def kernel(mllm_mask, byt5_mask):
    return _sc_call(mllm_mask, byt5_mask)


# speedup vs baseline: 1.1762x; 1.0021x over previous
import jax
import jax.numpy as jnp
from jax import lax
from jax.experimental import pallas as pl
from jax.experimental.pallas import tpu as pltpu, tpu_sc as plsc

_N = 1985
_NM = 1256
_L = 16
_NP = 1280
_NVP = _NP // _L
_NVF = 2048 // _L


def _sc_body(mllm_hbm, byt5_hbm, idx_hbm, zm_hbm, c_v, idx_v, zm_v):
    is_lead = jnp.logical_and(lax.axis_index("c") == 0, lax.axis_index("s") == 0)

    @pl.when(is_lead)
    def _():
        pltpu.sync_copy(mllm_hbm, c_v.at[pl.ds(0, 1000)])
        pltpu.sync_copy(byt5_hbm, c_v.at[pl.ds(1000, 256)])
        zeros = jnp.zeros((_L,), jnp.float32)
        c_v[pl.ds(_NM, _L)] = zeros
        c_v[pl.ds(_NP - _L, _L)] = zeros

        def sum_body(k, acc):
            return acc + c_v[pl.ds(k * _L, _L)]

        acc = lax.fori_loop(0, _NVP, sum_body, jnp.zeros((_L,), jnp.float32))
        n_valid = jnp.sum(acc)

        lane = lax.iota(jnp.int32, _L)
        lane_f = lane.astype(jnp.float32)

        def part_body(k, ones_before):
            c = c_v[pl.ds(k * _L, _L)]
            incl = plsc.cumsum(c)
            one_excl = ones_before + incl - c
            gi = k * _L + lane
            gi_f = gi.astype(jnp.float32)
            pos_f = jnp.where(c >= 0.5, one_excl, n_valid + gi_f - one_excl)
            plsc.store_scatter(idx_v, [pos_f.astype(jnp.int32)], gi)
            zm_v[pl.ds(k * _L, _L)] = (gi_f < n_valid).astype(jnp.float32)
            return ones_before + jnp.sum(c)

        lax.fori_loop(0, _NVP, part_body, jnp.float32(0.0))

        def fill_body(k, _):
            idx_v[pl.ds(k * _L, _L)] = k * _L + lane
            zm_v[pl.ds(k * _L, _L)] = zeros
            return 0

        lax.fori_loop(_NVP, _NVF, fill_body, 0)

        pltpu.sync_copy(idx_v.at[pl.ds(0, 1984)], idx_hbm.at[pl.ds(0, 1984)])
        pltpu.sync_copy(idx_v.at[pl.ds(1984, 1)], idx_hbm.at[pl.ds(1984, 1)])
        pltpu.sync_copy(zm_v.at[pl.ds(0, 1984)], zm_hbm.at[pl.ds(0, 1984)])
        pltpu.sync_copy(zm_v.at[pl.ds(1984, 1)], zm_hbm.at[pl.ds(1984, 1)])


_sc_call = pl.kernel(
    _sc_body,
    out_type=(
        jax.ShapeDtypeStruct((_N,), jnp.int32),
        jax.ShapeDtypeStruct((_N,), jnp.float32),
    ),
    mesh=plsc.VectorSubcoreMesh(core_axis_name="c", subcore_axis_name="s", num_cores=1, num_subcores=1),
    scratch_types=(
        pltpu.VMEM((_NP,), jnp.float32),
        pltpu.VMEM((2048,), jnp.int32),
        pltpu.VMEM((2048,), jnp.float32),
    ),
    compiler_params=pltpu.CompilerParams(needs_layout_passes=False),
)


def kernel(mllm_mask, byt5_mask):
    return _sc_call(mllm_mask, byt5_mask)
